# baseline (speedup 1.0000x reference)
import functools

import jax
import jax.numpy as jnp
from jax import lax
from jax.experimental import pallas as pl
from jax.experimental.pallas import tpu as pltpu
from jax.experimental.pallas import tpu_sc as plsc

F32 = jnp.float32

N_TOTAL = 10000
NP = 10240
N_FEAT = 128
HID = 64
N_EDGES = 320000
NC, NS = 2, 16
NW = NC * NS
CHUNK = 128
K_CH = 80
E_PAD = NW * K_CH * CHUNK
RPT = NP // NS
BATCH, WIN_IN, WIN_OUT = 5, 20, 5
LSTM_IN = 12800
LSTM_H = 32
NCLS = 10
BLK = 1024
NSLOT = 4
NAHEAD = 2


def _agg_pipeline(g_hbm, src_v, dst_v, bufs, gsems, ssems, acc, K):
    for b in range(NAHEAD):
        pltpu.async_copy(g_hbm.at[src_v.at[b]], bufs[b], gsems[b])

    def body(gi, carry):
        for b in range(NSLOT):
            j = gi * NSLOT + b
            b2 = (b + NAHEAD) % NSLOT
            pltpu.make_async_copy(g_hbm.at[src_v.at[j]], bufs[b], gsems[b]).wait()
            pltpu.async_copy(bufs[b], acc.at[dst_v.at[j]], ssems[b], add=True)

            @pl.when(j + NAHEAD < K)
            def _next():
                @pl.when(j >= NSLOT - NAHEAD)
                def _drain_prev():
                    pltpu.make_async_copy(
                        bufs[b2], acc.at[dst_v.at[j + NAHEAD - NSLOT]],
                        ssems[b2]).wait()
                pltpu.async_copy(g_hbm.at[src_v.at[j + NAHEAD]], bufs[b2], gsems[b2])
        return carry

    lax.fori_loop(0, K // NSLOT, body, 0)
    for j in range(K - NSLOT, K):
        b = j % NSLOT
        pltpu.make_async_copy(bufs[b], acc.at[dst_v.at[j]], ssems[b]).wait()

@functools.cache
def _deg_call():
    mesh = plsc.VectorSubcoreMesh(core_axis_name="c", subcore_axis_name="s")

    @functools.partial(
        pl.kernel,
        mesh=mesh,
        compiler_params=pltpu.CompilerParams(use_tc_tiling_on_sc=False),
        out_type=jax.ShapeDtypeStruct((NC, NP, 16), F32),
        scratch_types=[
            pltpu.VMEM((K_CH, CHUNK), jnp.int32),
            pltpu.VMEM((CHUNK, 16), F32),
            pltpu.VMEM((16, 16), F32),
            pltpu.VMEM_SHARED((NP, 16), F32),
        ],
    )
    def deg_kernel(dst_hbm, out_hbm, dst_v, ones_v, zt_v, acc):
        c = lax.axis_index("c")
        s = lax.axis_index("s")
        wid = s * NC + c
        pltpu.sync_copy(dst_hbm.at[wid], dst_v)
        for r in range(16):
            zt_v[r, :] = jnp.zeros((16,), F32)
        for r in range(CHUNK):
            ones_v[r, :] = jnp.ones((16,), F32)

        def zbody(i, carry):
            pltpu.sync_copy(zt_v, acc.at[pl.ds(s * RPT + i * 16, 16)])
            return carry

        lax.fori_loop(0, RPT // 16, zbody, 0)
        plsc.subcore_barrier()

        def sbody(j, carry):
            pltpu.sync_copy(ones_v, acc.at[dst_v.at[j]], add=True)
            return carry

        lax.fori_loop(0, K_CH, sbody, 0)
        plsc.subcore_barrier()
        pltpu.sync_copy(acc.at[pl.ds(s * RPT, RPT)],
                        out_hbm.at[c, pl.ds(s * RPT, RPT)])

    return deg_kernel


@functools.cache
def _agg_call(D):
    mesh = plsc.VectorSubcoreMesh(core_axis_name="c", subcore_axis_name="s")

    @functools.partial(
        pl.kernel,
        mesh=mesh,
        compiler_params=pltpu.CompilerParams(use_tc_tiling_on_sc=False),
        out_type=jax.ShapeDtypeStruct((NC, NP, D), F32),
        scratch_types=[
            pltpu.VMEM((K_CH, CHUNK), jnp.int32),
            pltpu.VMEM((K_CH, CHUNK), jnp.int32),
            pltpu.VMEM((NSLOT * CHUNK, D), F32),
            pltpu.VMEM((16, D), F32),
            pltpu.VMEM_SHARED((NP, D), F32),
        ] + [pltpu.SemaphoreType.DMA] * (2 * NSLOT),
    )
    def agg_kernel(g_hbm, src_hbm, dst_hbm, out_hbm,
                   src_v, dst_v, bufs_v, zt_v, acc, *sems):
        c = lax.axis_index("c")
        s = lax.axis_index("s")
        wid = s * NC + c
        pltpu.sync_copy(src_hbm.at[wid], src_v)
        pltpu.sync_copy(dst_hbm.at[wid], dst_v)
        for r in range(16):
            for q in range(D // 16):
                zt_v[r, pl.ds(q * 16, 16)] = jnp.zeros((16,), F32)

        def zbody(i, carry):
            pltpu.sync_copy(zt_v, acc.at[pl.ds(s * RPT + i * 16, 16)])
            return carry

        lax.fori_loop(0, RPT // 16, zbody, 0)
        plsc.subcore_barrier()

        bufs = [bufs_v.at[pl.ds(b * CHUNK, CHUNK)] for b in range(NSLOT)]
        _agg_pipeline(g_hbm.at[c], src_v, dst_v, bufs,
                      sems[:NSLOT], sems[NSLOT:], acc, K_CH)
        plsc.subcore_barrier()
        pltpu.sync_copy(acc.at[pl.ds(s * RPT, RPT)],
                        out_hbm.at[c, pl.ds(s * RPT, RPT)])

    return agg_kernel


K2 = 160


@functools.cache
def _agg_feat_call():
    mesh = plsc.VectorSubcoreMesh(core_axis_name="c", subcore_axis_name="s")

    @functools.partial(
        pl.kernel,
        mesh=mesh,
        compiler_params=pltpu.CompilerParams(use_tc_tiling_on_sc=False),
        out_type=jax.ShapeDtypeStruct((NC, NP, HID), F32),
        scratch_types=[
            pltpu.VMEM((K2, CHUNK), jnp.int32),
            pltpu.VMEM((K2, CHUNK), jnp.int32),
            pltpu.VMEM((NSLOT * CHUNK, HID), F32),
            pltpu.VMEM((16, HID), F32),
            pltpu.VMEM_SHARED((NP, HID), F32),
        ] + [pltpu.SemaphoreType.DMA] * (2 * NSLOT),
    )
    def agg2_kernel(g_hbm, src_hbm, dst_hbm, out_hbm,
                    src_v, dst_v, bufs_v, zt_v, acc, *sems):
        c = lax.axis_index("c")
        s = lax.axis_index("s")
        pltpu.sync_copy(src_hbm.at[s], src_v)
        pltpu.sync_copy(dst_hbm.at[s], dst_v)
        for r in range(16):
            for q in range(HID // 16):
                zt_v[r, pl.ds(q * 16, 16)] = jnp.zeros((16,), F32)

        def zbody(i, carry):
            pltpu.sync_copy(zt_v, acc.at[pl.ds(s * RPT + i * 16, 16)])
            return carry

        lax.fori_loop(0, RPT // 16, zbody, 0)
        plsc.subcore_barrier()

        bufs = [bufs_v.at[pl.ds(b * CHUNK, CHUNK)] for b in range(NSLOT)]
        _agg_pipeline(g_hbm.at[c], src_v, dst_v, bufs,
                      sems[:NSLOT], sems[NSLOT:], acc, K2)
        plsc.subcore_barrier()
        pltpu.sync_copy(acc.at[pl.ds(s * RPT, RPT)],
                        out_hbm.at[c, pl.ds(s * RPT, RPT)])

    return agg2_kernel


def _sc_degree(dstp):
    return _deg_call()(dstp)


def _sc_aggregate(g, srcp, dstp, D):
    return _agg_call(D)(g, srcp, dstp)


def _sc_aggregate_feat(g2d, srcq, dstq):
    return _agg_feat_call()(g2d, srcq, dstq)


def _dinv_col(deg_ref):
    d = deg_ref[0] + deg_ref[1]
    return lax.rsqrt(d[:, 0:1] + 1.0)


def _prep_body(x_ref, w_ref, deg_ref, h1_ref, g1_ref):
    h = jnp.dot(x_ref[...], w_ref[...], preferred_element_type=F32)
    h1_ref[...] = h
    g1 = h * _dinv_col(deg_ref)
    g1_ref[0] = g1
    g1_ref[1] = g1


def _tc_prep(xp, W1, deg_p):
    return pl.pallas_call(
        _prep_body,
        grid=(NP // BLK,),
        in_specs=[
            pl.BlockSpec((BLK, N_FEAT), lambda i: (i, 0)),
            pl.BlockSpec((N_FEAT, HID), lambda i: (0, 0)),
            pl.BlockSpec((NC, BLK, 16), lambda i: (0, i, 0)),
        ],
        out_specs=[
            pl.BlockSpec((BLK, HID), lambda i: (i, 0)),
            pl.BlockSpec((NC, BLK, HID), lambda i: (0, i, 0)),
        ],
        out_shape=[
            jax.ShapeDtypeStruct((NP, HID), F32),
            jax.ShapeDtypeStruct((NC, NP, HID), F32),
        ],
    )(xp, W1, deg_p)


def _mid_body(agg_ref, h1_ref, deg_ref, b1_ref, w2_ref, h2_ref, g2_ref):
    dv = _dinv_col(deg_ref)
    agg = agg_ref[0] + agg_ref[1]
    out1 = jnp.maximum(dv * agg + dv * dv * h1_ref[...] + b1_ref[...], 0.0)
    h2 = jnp.dot(out1, w2_ref[...], preferred_element_type=F32)
    h2_ref[...] = h2
    g2 = h2 * dv
    g2_ref[0] = g2[:, 0:HID]
    g2_ref[1] = g2[:, HID:N_FEAT]


def _tc_mid(agg1, H1, deg_p, b1, W2):
    return pl.pallas_call(
        _mid_body,
        grid=(NP // BLK,),
        in_specs=[
            pl.BlockSpec((NC, BLK, HID), lambda i: (0, i, 0)),
            pl.BlockSpec((BLK, HID), lambda i: (i, 0)),
            pl.BlockSpec((NC, BLK, 16), lambda i: (0, i, 0)),
            pl.BlockSpec((1, HID), lambda i: (0, 0)),
            pl.BlockSpec((HID, N_FEAT), lambda i: (0, 0)),
        ],
        out_specs=[
            pl.BlockSpec((BLK, N_FEAT), lambda i: (i, 0)),
            pl.BlockSpec((NC, BLK, HID), lambda i: (0, i, 0)),
        ],
        out_shape=[
            jax.ShapeDtypeStruct((NP, N_FEAT), F32),
            jax.ShapeDtypeStruct((NC, NP, HID), F32),
        ],
    )(agg1, H1, deg_p, b1, W2)


def _post_body(agg_ref, h2_ref, deg_ref, b2_ref, out_ref):
    dv = _dinv_col(deg_ref)
    agg = jnp.concatenate([agg_ref[0], agg_ref[1]], axis=1)
    out_ref[...] = jnp.maximum(dv * agg + dv * dv * h2_ref[...] + b2_ref[...], 0.0)


def _tc_post(agg2, H2, deg_p, b2):
    return pl.pallas_call(
        _post_body,
        grid=(NP // BLK,),
        in_specs=[
            pl.BlockSpec((NC, BLK, HID), lambda i: (0, i, 0)),
            pl.BlockSpec((BLK, N_FEAT), lambda i: (i, 0)),
            pl.BlockSpec((NC, BLK, 16), lambda i: (0, i, 0)),
            pl.BlockSpec((1, N_FEAT), lambda i: (0, 0)),
        ],
        out_specs=pl.BlockSpec((BLK, N_FEAT), lambda i: (i, 0)),
        out_shape=jax.ShapeDtypeStruct((NP, N_FEAT), F32),
    )(agg2, H2, deg_p, b2)


def _lstm_in_body(x_ref, w_ref, bi_ref, bh_ref, out_ref):
    k = pl.program_id(0)

    @pl.when(k == 0)
    def _init():
        out_ref[...] = jnp.zeros_like(out_ref)

    out_ref[...] += jnp.dot(x_ref[...], w_ref[...], preferred_element_type=F32)

    @pl.when(k == pl.num_programs(0) - 1)
    def _bias():
        out_ref[...] += bi_ref[...] + bh_ref[...]


def _tc_lstm_in(xl, W_ihT, bi, bh):
    kb = 512
    return pl.pallas_call(
        _lstm_in_body,
        grid=(LSTM_IN // kb,),
        in_specs=[
            pl.BlockSpec((104, kb), lambda k: (0, k)),
            pl.BlockSpec((kb, 4 * LSTM_H), lambda k: (k, 0)),
            pl.BlockSpec((1, 4 * LSTM_H), lambda k: (0, 0)),
            pl.BlockSpec((1, 4 * LSTM_H), lambda k: (0, 0)),
        ],
        out_specs=pl.BlockSpec((104, 4 * LSTM_H), lambda k: (0, 0)),
        out_shape=jax.ShapeDtypeStruct((104, 4 * LSTM_H), F32),
    )(xl, W_ihT, bi, bh)


def _lstm_fc_body(g_ref, whh_ref, fc1w_ref, fc1b_ref, fc2w_ref, fc2b_ref, out_ref):
    whh = whh_ref[...]
    h = jnp.zeros((BATCH, LSTM_H), F32)
    cst = jnp.zeros((BATCH, LSTM_H), F32)
    hs = []
    for t in range(WIN_IN):
        gt = jnp.concatenate(
            [g_ref[pl.ds(b * WIN_IN + t, 1), :] for b in range(BATCH)], axis=0)
        gates = gt + jnp.dot(h, whh, preferred_element_type=F32)
        i_g = jax.nn.sigmoid(gates[:, 0:LSTM_H])
        f_g = jax.nn.sigmoid(gates[:, LSTM_H:2 * LSTM_H])
        g_g = jnp.tanh(gates[:, 2 * LSTM_H:3 * LSTM_H])
        o_g = jax.nn.sigmoid(gates[:, 3 * LSTM_H:4 * LSTM_H])
        cst = f_g * cst + i_g * g_g
        h = o_g * jnp.tanh(cst)
        if t >= WIN_IN - WIN_OUT:
            hs.append(h)
    hcat = jnp.concatenate(hs, axis=0)
    z = jnp.maximum(
        jnp.dot(hcat, fc1w_ref[...], preferred_element_type=F32) + fc1b_ref[...], 0.0)
    out_ref[...] = jnp.dot(z, fc2w_ref[...], preferred_element_type=F32) + fc2b_ref[...]


def _tc_lstm_fc(G, W_hhT, fc1_W, fc1_b, fc2_W, fc2_b):
    return pl.pallas_call(
        _lstm_fc_body,
        out_shape=jax.ShapeDtypeStruct((BATCH * WIN_OUT, NCLS), F32),
    )(G, W_hhT, fc1_W, fc1_b, fc2_W, fc2_b)


def kernel(x, edge_index, W1, b1, W2, b2, W_ih, W_hh, b_ih, b_hh,
           fc1_W, fc1_b, fc2_W, fc2_b):
    xp = jnp.pad(x, ((0, NP - N_TOTAL), (0, 0)))
    pad = jnp.full((E_PAD - N_EDGES,), N_TOTAL, jnp.int32)
    src_f = jnp.concatenate([edge_index[0], pad])
    dst_f = jnp.concatenate([edge_index[1], pad])
    srcp = src_f.reshape(NW, K_CH, CHUNK)
    dstp = dst_f.reshape(NW, K_CH, CHUNK)
    srcq = src_f.reshape(NS, K2, CHUNK)
    dstq = dst_f.reshape(NS, K2, CHUNK)

    deg_p = _sc_degree(dstp)

    H1, g1 = _tc_prep(xp, W1, deg_p)
    agg1 = _sc_aggregate(g1, srcp, dstp, HID)
    H2, g2h = _tc_mid(agg1, H1, deg_p, b1.reshape(1, HID), W2)
    agg2 = _sc_aggregate_feat(g2h, srcq, dstq)
    out2 = _tc_post(agg2, H2, deg_p, b2.reshape(1, N_FEAT))

    xl = out2[:N_TOTAL].reshape(BATCH * WIN_IN, LSTM_IN)
    xl = jnp.pad(xl, ((0, 4), (0, 0)))
    G = _tc_lstm_in(xl, W_ih.T, b_ih.reshape(1, -1), b_hh.reshape(1, -1))
    out_t = _tc_lstm_fc(G, W_hh.T, fc1_W, fc1_b.reshape(1, -1),
                        fc2_W, fc2_b.reshape(1, -1))
    return (out_t.reshape(WIN_OUT, BATCH, NCLS)
            .transpose(1, 0, 2).reshape(BATCH * WIN_OUT, NCLS))


# speedup vs baseline: 1.0141x; 1.0141x over previous
import functools

import jax
import jax.numpy as jnp
from jax import lax
from jax.experimental import pallas as pl
from jax.experimental.pallas import tpu as pltpu
from jax.experimental.pallas import tpu_sc as plsc

F32 = jnp.float32

N_TOTAL = 10000
NP = 10240
N_FEAT = 128
HID = 64
N_EDGES = 320000
NC, NS = 2, 16
NW = NC * NS
CHUNK = 128
K_CH = 80
E_PAD = NW * K_CH * CHUNK
RPT = NP // NS
BATCH, WIN_IN, WIN_OUT = 5, 20, 5
LSTM_IN = 12800
LSTM_H = 32
NCLS = 10
BLK = 1024
NSLOT = 5
NAHEAD = 3


def _agg_pipeline(g_hbm, src_v, dst_v, bufs, gsems, ssems, acc, K):
    for b in range(NAHEAD):
        pltpu.async_copy(g_hbm.at[src_v.at[b]], bufs[b], gsems[b])

    def body(gi, carry):
        for b in range(NSLOT):
            j = gi * NSLOT + b
            b2 = (b + NAHEAD) % NSLOT
            pltpu.make_async_copy(g_hbm.at[src_v.at[j]], bufs[b], gsems[b]).wait()
            pltpu.async_copy(bufs[b], acc.at[dst_v.at[j]], ssems[b], add=True)

            @pl.when(j + NAHEAD < K)
            def _next():
                @pl.when(j >= NSLOT - NAHEAD)
                def _drain_prev():
                    pltpu.make_async_copy(
                        bufs[b2], acc.at[dst_v.at[j + NAHEAD - NSLOT]],
                        ssems[b2]).wait()
                pltpu.async_copy(g_hbm.at[src_v.at[j + NAHEAD]], bufs[b2], gsems[b2])
        return carry

    lax.fori_loop(0, K // NSLOT, body, 0)
    for j in range(K - NSLOT, K):
        b = j % NSLOT
        pltpu.make_async_copy(bufs[b], acc.at[dst_v.at[j]], ssems[b]).wait()

@functools.cache
def _deg_call():
    mesh = plsc.VectorSubcoreMesh(core_axis_name="c", subcore_axis_name="s")

    @functools.partial(
        pl.kernel,
        mesh=mesh,
        compiler_params=pltpu.CompilerParams(use_tc_tiling_on_sc=False),
        out_type=jax.ShapeDtypeStruct((NC, NP, 16), F32),
        scratch_types=[
            pltpu.VMEM((K_CH, CHUNK), jnp.int32),
            pltpu.VMEM((CHUNK, 16), F32),
            pltpu.VMEM((16, 16), F32),
            pltpu.VMEM_SHARED((NP, 16), F32),
        ],
    )
    def deg_kernel(dst_hbm, out_hbm, dst_v, ones_v, zt_v, acc):
        c = lax.axis_index("c")
        s = lax.axis_index("s")
        wid = s * NC + c
        pltpu.sync_copy(dst_hbm.at[wid], dst_v)
        for r in range(16):
            zt_v[r, :] = jnp.zeros((16,), F32)
        for r in range(CHUNK):
            ones_v[r, :] = jnp.ones((16,), F32)

        def zbody(i, carry):
            pltpu.sync_copy(zt_v, acc.at[pl.ds(s * RPT + i * 16, 16)])
            return carry

        lax.fori_loop(0, RPT // 16, zbody, 0)
        plsc.subcore_barrier()

        def sbody(j, carry):
            pltpu.sync_copy(ones_v, acc.at[dst_v.at[j]], add=True)
            return carry

        lax.fori_loop(0, K_CH, sbody, 0)
        plsc.subcore_barrier()
        pltpu.sync_copy(acc.at[pl.ds(s * RPT, RPT)],
                        out_hbm.at[c, pl.ds(s * RPT, RPT)])

    return deg_kernel


@functools.cache
def _agg_call(D):
    mesh = plsc.VectorSubcoreMesh(core_axis_name="c", subcore_axis_name="s")

    @functools.partial(
        pl.kernel,
        mesh=mesh,
        compiler_params=pltpu.CompilerParams(use_tc_tiling_on_sc=False),
        out_type=jax.ShapeDtypeStruct((NC, NP, D), F32),
        scratch_types=[
            pltpu.VMEM((K_CH, CHUNK), jnp.int32),
            pltpu.VMEM((K_CH, CHUNK), jnp.int32),
            pltpu.VMEM((NSLOT * CHUNK, D), F32),
            pltpu.VMEM((16, D), F32),
            pltpu.VMEM_SHARED((NP, D), F32),
        ] + [pltpu.SemaphoreType.DMA] * (2 * NSLOT),
    )
    def agg_kernel(g_hbm, src_hbm, dst_hbm, out_hbm,
                   src_v, dst_v, bufs_v, zt_v, acc, *sems):
        c = lax.axis_index("c")
        s = lax.axis_index("s")
        wid = s * NC + c
        pltpu.sync_copy(src_hbm.at[wid], src_v)
        pltpu.sync_copy(dst_hbm.at[wid], dst_v)
        for r in range(16):
            for q in range(D // 16):
                zt_v[r, pl.ds(q * 16, 16)] = jnp.zeros((16,), F32)

        def zbody(i, carry):
            pltpu.sync_copy(zt_v, acc.at[pl.ds(s * RPT + i * 16, 16)])
            return carry

        lax.fori_loop(0, RPT // 16, zbody, 0)
        plsc.subcore_barrier()

        bufs = [bufs_v.at[pl.ds(b * CHUNK, CHUNK)] for b in range(NSLOT)]
        _agg_pipeline(g_hbm.at[c], src_v, dst_v, bufs,
                      sems[:NSLOT], sems[NSLOT:], acc, K_CH)
        plsc.subcore_barrier()
        pltpu.sync_copy(acc.at[pl.ds(s * RPT, RPT)],
                        out_hbm.at[c, pl.ds(s * RPT, RPT)])

    return agg_kernel


K2 = 160


@functools.cache
def _agg_feat_call():
    mesh = plsc.VectorSubcoreMesh(core_axis_name="c", subcore_axis_name="s")

    @functools.partial(
        pl.kernel,
        mesh=mesh,
        compiler_params=pltpu.CompilerParams(use_tc_tiling_on_sc=False),
        out_type=jax.ShapeDtypeStruct((NC, NP, HID), F32),
        scratch_types=[
            pltpu.VMEM((K2, CHUNK), jnp.int32),
            pltpu.VMEM((K2, CHUNK), jnp.int32),
            pltpu.VMEM((NSLOT * CHUNK, HID), F32),
            pltpu.VMEM((16, HID), F32),
            pltpu.VMEM_SHARED((NP, HID), F32),
        ] + [pltpu.SemaphoreType.DMA] * (2 * NSLOT),
    )
    def agg2_kernel(g_hbm, src_hbm, dst_hbm, out_hbm,
                    src_v, dst_v, bufs_v, zt_v, acc, *sems):
        c = lax.axis_index("c")
        s = lax.axis_index("s")
        pltpu.sync_copy(src_hbm.at[s], src_v)
        pltpu.sync_copy(dst_hbm.at[s], dst_v)
        for r in range(16):
            for q in range(HID // 16):
                zt_v[r, pl.ds(q * 16, 16)] = jnp.zeros((16,), F32)

        def zbody(i, carry):
            pltpu.sync_copy(zt_v, acc.at[pl.ds(s * RPT + i * 16, 16)])
            return carry

        lax.fori_loop(0, RPT // 16, zbody, 0)
        plsc.subcore_barrier()

        bufs = [bufs_v.at[pl.ds(b * CHUNK, CHUNK)] for b in range(NSLOT)]
        _agg_pipeline(g_hbm.at[c], src_v, dst_v, bufs,
                      sems[:NSLOT], sems[NSLOT:], acc, K2)
        plsc.subcore_barrier()
        pltpu.sync_copy(acc.at[pl.ds(s * RPT, RPT)],
                        out_hbm.at[c, pl.ds(s * RPT, RPT)])

    return agg2_kernel


def _sc_degree(dstp):
    return _deg_call()(dstp)


def _sc_aggregate(g, srcp, dstp, D):
    return _agg_call(D)(g, srcp, dstp)


def _sc_aggregate_feat(g2d, srcq, dstq):
    return _agg_feat_call()(g2d, srcq, dstq)


def _dinv_col(deg_ref):
    d = deg_ref[0] + deg_ref[1]
    return lax.rsqrt(d[:, 0:1] + 1.0)


def _prep_body(x_ref, w_ref, deg_ref, h1_ref, g1_ref):
    h = jnp.dot(x_ref[...], w_ref[...], preferred_element_type=F32)
    h1_ref[...] = h
    g1 = h * _dinv_col(deg_ref)
    g1_ref[0] = g1
    g1_ref[1] = g1


def _tc_prep(xp, W1, deg_p):
    return pl.pallas_call(
        _prep_body,
        grid=(NP // BLK,),
        in_specs=[
            pl.BlockSpec((BLK, N_FEAT), lambda i: (i, 0)),
            pl.BlockSpec((N_FEAT, HID), lambda i: (0, 0)),
            pl.BlockSpec((NC, BLK, 16), lambda i: (0, i, 0)),
        ],
        out_specs=[
            pl.BlockSpec((BLK, HID), lambda i: (i, 0)),
            pl.BlockSpec((NC, BLK, HID), lambda i: (0, i, 0)),
        ],
        out_shape=[
            jax.ShapeDtypeStruct((NP, HID), F32),
            jax.ShapeDtypeStruct((NC, NP, HID), F32),
        ],
    )(xp, W1, deg_p)


def _mid_body(agg_ref, h1_ref, deg_ref, b1_ref, w2_ref, h2_ref, g2_ref):
    dv = _dinv_col(deg_ref)
    agg = agg_ref[0] + agg_ref[1]
    out1 = jnp.maximum(dv * agg + dv * dv * h1_ref[...] + b1_ref[...], 0.0)
    h2 = jnp.dot(out1, w2_ref[...], preferred_element_type=F32)
    h2_ref[...] = h2
    g2 = h2 * dv
    g2_ref[0] = g2[:, 0:HID]
    g2_ref[1] = g2[:, HID:N_FEAT]


def _tc_mid(agg1, H1, deg_p, b1, W2):
    return pl.pallas_call(
        _mid_body,
        grid=(NP // BLK,),
        in_specs=[
            pl.BlockSpec((NC, BLK, HID), lambda i: (0, i, 0)),
            pl.BlockSpec((BLK, HID), lambda i: (i, 0)),
            pl.BlockSpec((NC, BLK, 16), lambda i: (0, i, 0)),
            pl.BlockSpec((1, HID), lambda i: (0, 0)),
            pl.BlockSpec((HID, N_FEAT), lambda i: (0, 0)),
        ],
        out_specs=[
            pl.BlockSpec((BLK, N_FEAT), lambda i: (i, 0)),
            pl.BlockSpec((NC, BLK, HID), lambda i: (0, i, 0)),
        ],
        out_shape=[
            jax.ShapeDtypeStruct((NP, N_FEAT), F32),
            jax.ShapeDtypeStruct((NC, NP, HID), F32),
        ],
    )(agg1, H1, deg_p, b1, W2)


def _post_body(agg_ref, h2_ref, deg_ref, b2_ref, out_ref):
    dv = _dinv_col(deg_ref)
    agg = jnp.concatenate([agg_ref[0], agg_ref[1]], axis=1)
    out_ref[...] = jnp.maximum(dv * agg + dv * dv * h2_ref[...] + b2_ref[...], 0.0)


def _tc_post(agg2, H2, deg_p, b2):
    return pl.pallas_call(
        _post_body,
        grid=(NP // BLK,),
        in_specs=[
            pl.BlockSpec((NC, BLK, HID), lambda i: (0, i, 0)),
            pl.BlockSpec((BLK, N_FEAT), lambda i: (i, 0)),
            pl.BlockSpec((NC, BLK, 16), lambda i: (0, i, 0)),
            pl.BlockSpec((1, N_FEAT), lambda i: (0, 0)),
        ],
        out_specs=pl.BlockSpec((BLK, N_FEAT), lambda i: (i, 0)),
        out_shape=jax.ShapeDtypeStruct((NP, N_FEAT), F32),
    )(agg2, H2, deg_p, b2)


def _lstm_in_body(x_ref, w_ref, bi_ref, bh_ref, out_ref):
    k = pl.program_id(0)

    @pl.when(k == 0)
    def _init():
        out_ref[...] = jnp.zeros_like(out_ref)

    out_ref[...] += jnp.dot(x_ref[...], w_ref[...], preferred_element_type=F32)

    @pl.when(k == pl.num_programs(0) - 1)
    def _bias():
        out_ref[...] += bi_ref[...] + bh_ref[...]


def _tc_lstm_in(xl, W_ihT, bi, bh):
    kb = 512
    return pl.pallas_call(
        _lstm_in_body,
        grid=(LSTM_IN // kb,),
        in_specs=[
            pl.BlockSpec((104, kb), lambda k: (0, k)),
            pl.BlockSpec((kb, 4 * LSTM_H), lambda k: (k, 0)),
            pl.BlockSpec((1, 4 * LSTM_H), lambda k: (0, 0)),
            pl.BlockSpec((1, 4 * LSTM_H), lambda k: (0, 0)),
        ],
        out_specs=pl.BlockSpec((104, 4 * LSTM_H), lambda k: (0, 0)),
        out_shape=jax.ShapeDtypeStruct((104, 4 * LSTM_H), F32),
    )(xl, W_ihT, bi, bh)


def _lstm_fc_body(g_ref, whh_ref, fc1w_ref, fc1b_ref, fc2w_ref, fc2b_ref, out_ref):
    whh = whh_ref[...]
    h = jnp.zeros((BATCH, LSTM_H), F32)
    cst = jnp.zeros((BATCH, LSTM_H), F32)
    hs = []
    for t in range(WIN_IN):
        gt = jnp.concatenate(
            [g_ref[pl.ds(b * WIN_IN + t, 1), :] for b in range(BATCH)], axis=0)
        gates = gt + jnp.dot(h, whh, preferred_element_type=F32)
        i_g = jax.nn.sigmoid(gates[:, 0:LSTM_H])
        f_g = jax.nn.sigmoid(gates[:, LSTM_H:2 * LSTM_H])
        g_g = jnp.tanh(gates[:, 2 * LSTM_H:3 * LSTM_H])
        o_g = jax.nn.sigmoid(gates[:, 3 * LSTM_H:4 * LSTM_H])
        cst = f_g * cst + i_g * g_g
        h = o_g * jnp.tanh(cst)
        if t >= WIN_IN - WIN_OUT:
            hs.append(h)
    hcat = jnp.concatenate(hs, axis=0)
    z = jnp.maximum(
        jnp.dot(hcat, fc1w_ref[...], preferred_element_type=F32) + fc1b_ref[...], 0.0)
    out_ref[...] = jnp.dot(z, fc2w_ref[...], preferred_element_type=F32) + fc2b_ref[...]


def _tc_lstm_fc(G, W_hhT, fc1_W, fc1_b, fc2_W, fc2_b):
    return pl.pallas_call(
        _lstm_fc_body,
        out_shape=jax.ShapeDtypeStruct((BATCH * WIN_OUT, NCLS), F32),
    )(G, W_hhT, fc1_W, fc1_b, fc2_W, fc2_b)


def kernel(x, edge_index, W1, b1, W2, b2, W_ih, W_hh, b_ih, b_hh,
           fc1_W, fc1_b, fc2_W, fc2_b):
    xp = jnp.pad(x, ((0, NP - N_TOTAL), (0, 0)))
    pad = jnp.full((E_PAD - N_EDGES,), N_TOTAL, jnp.int32)
    src_f = jnp.concatenate([edge_index[0], pad])
    dst_f = jnp.concatenate([edge_index[1], pad])
    srcp = src_f.reshape(NW, K_CH, CHUNK)
    dstp = dst_f.reshape(NW, K_CH, CHUNK)
    srcq = src_f.reshape(NS, K2, CHUNK)
    dstq = dst_f.reshape(NS, K2, CHUNK)

    deg_p = _sc_degree(dstp)

    H1, g1 = _tc_prep(xp, W1, deg_p)
    agg1 = _sc_aggregate(g1, srcp, dstp, HID)
    H2, g2h = _tc_mid(agg1, H1, deg_p, b1.reshape(1, HID), W2)
    agg2 = _sc_aggregate_feat(g2h, srcq, dstq)
    out2 = _tc_post(agg2, H2, deg_p, b2.reshape(1, N_FEAT))

    xl = out2[:N_TOTAL].reshape(BATCH * WIN_IN, LSTM_IN)
    xl = jnp.pad(xl, ((0, 4), (0, 0)))
    G = _tc_lstm_in(xl, W_ih.T, b_ih.reshape(1, -1), b_hh.reshape(1, -1))
    out_t = _tc_lstm_fc(G, W_hh.T, fc1_W, fc1_b.reshape(1, -1),
                        fc2_W, fc2_b.reshape(1, -1))
    return (out_t.reshape(WIN_OUT, BATCH, NCLS)
            .transpose(1, 0, 2).reshape(BATCH * WIN_OUT, NCLS))


# speedup vs baseline: 1.0155x; 1.0014x over previous
import functools

import jax
import jax.numpy as jnp
from jax import lax
from jax.experimental import pallas as pl
from jax.experimental.pallas import tpu as pltpu
from jax.experimental.pallas import tpu_sc as plsc

F32 = jnp.float32

N_TOTAL = 10000
NP = 10240
N_FEAT = 128
HID = 64
N_EDGES = 320000
NC, NS = 2, 16
NW = NC * NS
CHUNK = 128
K_CH = 80
E_PAD = NW * K_CH * CHUNK
RPT = NP // NS
BATCH, WIN_IN, WIN_OUT = 5, 20, 5
LSTM_IN = 12800
LSTM_H = 32
NCLS = 10
BLK = 1024
NSLOT = 5
NAHEAD = 4


def _agg_pipeline(g_hbm, src_v, dst_v, bufs, gsems, ssems, acc, K):
    for b in range(NAHEAD):
        pltpu.async_copy(g_hbm.at[src_v.at[b]], bufs[b], gsems[b])

    def body(gi, carry):
        for b in range(NSLOT):
            j = gi * NSLOT + b
            b2 = (b + NAHEAD) % NSLOT
            pltpu.make_async_copy(g_hbm.at[src_v.at[j]], bufs[b], gsems[b]).wait()
            pltpu.async_copy(bufs[b], acc.at[dst_v.at[j]], ssems[b], add=True)

            @pl.when(j + NAHEAD < K)
            def _next():
                @pl.when(j >= NSLOT - NAHEAD)
                def _drain_prev():
                    pltpu.make_async_copy(
                        bufs[b2], acc.at[dst_v.at[j + NAHEAD - NSLOT]],
                        ssems[b2]).wait()
                pltpu.async_copy(g_hbm.at[src_v.at[j + NAHEAD]], bufs[b2], gsems[b2])
        return carry

    lax.fori_loop(0, K // NSLOT, body, 0)
    for j in range(K - NSLOT, K):
        b = j % NSLOT
        pltpu.make_async_copy(bufs[b], acc.at[dst_v.at[j]], ssems[b]).wait()

@functools.cache
def _deg_call():
    mesh = plsc.VectorSubcoreMesh(core_axis_name="c", subcore_axis_name="s")

    @functools.partial(
        pl.kernel,
        mesh=mesh,
        compiler_params=pltpu.CompilerParams(use_tc_tiling_on_sc=False),
        out_type=jax.ShapeDtypeStruct((NC, NP, 16), F32),
        scratch_types=[
            pltpu.VMEM((K_CH, CHUNK), jnp.int32),
            pltpu.VMEM((CHUNK, 16), F32),
            pltpu.VMEM((16, 16), F32),
            pltpu.VMEM_SHARED((NP, 16), F32),
        ],
    )
    def deg_kernel(dst_hbm, out_hbm, dst_v, ones_v, zt_v, acc):
        c = lax.axis_index("c")
        s = lax.axis_index("s")
        wid = s * NC + c
        pltpu.sync_copy(dst_hbm.at[wid], dst_v)
        for r in range(16):
            zt_v[r, :] = jnp.zeros((16,), F32)
        for r in range(CHUNK):
            ones_v[r, :] = jnp.ones((16,), F32)

        def zbody(i, carry):
            pltpu.sync_copy(zt_v, acc.at[pl.ds(s * RPT + i * 16, 16)])
            return carry

        lax.fori_loop(0, RPT // 16, zbody, 0)
        plsc.subcore_barrier()

        def sbody(j, carry):
            pltpu.sync_copy(ones_v, acc.at[dst_v.at[j]], add=True)
            return carry

        lax.fori_loop(0, K_CH, sbody, 0)
        plsc.subcore_barrier()
        pltpu.sync_copy(acc.at[pl.ds(s * RPT, RPT)],
                        out_hbm.at[c, pl.ds(s * RPT, RPT)])

    return deg_kernel


@functools.cache
def _agg_call(D):
    mesh = plsc.VectorSubcoreMesh(core_axis_name="c", subcore_axis_name="s")

    @functools.partial(
        pl.kernel,
        mesh=mesh,
        compiler_params=pltpu.CompilerParams(use_tc_tiling_on_sc=False),
        out_type=jax.ShapeDtypeStruct((NC, NP, D), F32),
        scratch_types=[
            pltpu.VMEM((K_CH, CHUNK), jnp.int32),
            pltpu.VMEM((K_CH, CHUNK), jnp.int32),
            pltpu.VMEM((NSLOT * CHUNK, D), F32),
            pltpu.VMEM((16, D), F32),
            pltpu.VMEM_SHARED((NP, D), F32),
        ] + [pltpu.SemaphoreType.DMA] * (2 * NSLOT),
    )
    def agg_kernel(g_hbm, src_hbm, dst_hbm, out_hbm,
                   src_v, dst_v, bufs_v, zt_v, acc, *sems):
        c = lax.axis_index("c")
        s = lax.axis_index("s")
        wid = c * NS + s
        pltpu.sync_copy(src_hbm.at[wid], src_v)
        pltpu.sync_copy(dst_hbm.at[wid], dst_v)
        for r in range(16):
            for q in range(D // 16):
                zt_v[r, pl.ds(q * 16, 16)] = jnp.zeros((16,), F32)

        def zbody(i, carry):
            pltpu.sync_copy(zt_v, acc.at[pl.ds(s * RPT + i * 16, 16)])
            return carry

        lax.fori_loop(0, RPT // 16, zbody, 0)
        plsc.subcore_barrier()

        bufs = [bufs_v.at[pl.ds(b * CHUNK, CHUNK)] for b in range(NSLOT)]
        _agg_pipeline(g_hbm.at[c], src_v, dst_v, bufs,
                      sems[:NSLOT], sems[NSLOT:], acc, K_CH)
        plsc.subcore_barrier()
        pltpu.sync_copy(acc.at[pl.ds(s * RPT, RPT)],
                        out_hbm.at[c, pl.ds(s * RPT, RPT)])

    return agg_kernel


K2 = 160


@functools.cache
def _agg_feat_call():
    mesh = plsc.VectorSubcoreMesh(core_axis_name="c", subcore_axis_name="s")

    @functools.partial(
        pl.kernel,
        mesh=mesh,
        compiler_params=pltpu.CompilerParams(use_tc_tiling_on_sc=False),
        out_type=jax.ShapeDtypeStruct((NC, NP, HID), F32),
        scratch_types=[
            pltpu.VMEM((K2, CHUNK), jnp.int32),
            pltpu.VMEM((K2, CHUNK), jnp.int32),
            pltpu.VMEM((NSLOT * CHUNK, HID), F32),
            pltpu.VMEM((16, HID), F32),
            pltpu.VMEM_SHARED((NP, HID), F32),
        ] + [pltpu.SemaphoreType.DMA] * (2 * NSLOT),
    )
    def agg2_kernel(g_hbm, src_hbm, dst_hbm, out_hbm,
                    src_v, dst_v, bufs_v, zt_v, acc, *sems):
        c = lax.axis_index("c")
        s = lax.axis_index("s")
        pltpu.sync_copy(src_hbm.at[s], src_v)
        pltpu.sync_copy(dst_hbm.at[s], dst_v)
        for r in range(16):
            for q in range(HID // 16):
                zt_v[r, pl.ds(q * 16, 16)] = jnp.zeros((16,), F32)

        def zbody(i, carry):
            pltpu.sync_copy(zt_v, acc.at[pl.ds(s * RPT + i * 16, 16)])
            return carry

        lax.fori_loop(0, RPT // 16, zbody, 0)
        plsc.subcore_barrier()

        bufs = [bufs_v.at[pl.ds(b * CHUNK, CHUNK)] for b in range(NSLOT)]
        _agg_pipeline(g_hbm.at[c], src_v, dst_v, bufs,
                      sems[:NSLOT], sems[NSLOT:], acc, K2)
        plsc.subcore_barrier()
        pltpu.sync_copy(acc.at[pl.ds(s * RPT, RPT)],
                        out_hbm.at[c, pl.ds(s * RPT, RPT)])

    return agg2_kernel


def _sc_degree(dstp):
    return _deg_call()(dstp)


def _sc_aggregate(g, srcp, dstp, D):
    return _agg_call(D)(g, srcp, dstp)


def _sc_aggregate_feat(g2d, srcq, dstq):
    return _agg_feat_call()(g2d, srcq, dstq)


def _dinv_col(deg_ref):
    d = deg_ref[0] + deg_ref[1]
    return lax.rsqrt(d[:, 0:1] + 1.0)


def _prep_body(x_ref, w_ref, deg_ref, h1_ref, g1_ref):
    h = jnp.dot(x_ref[...], w_ref[...], preferred_element_type=F32)
    h1_ref[...] = h
    g1 = h * _dinv_col(deg_ref)
    g1_ref[0] = g1
    g1_ref[1] = g1


def _tc_prep(xp, W1, deg_p):
    return pl.pallas_call(
        _prep_body,
        grid=(NP // BLK,),
        in_specs=[
            pl.BlockSpec((BLK, N_FEAT), lambda i: (i, 0)),
            pl.BlockSpec((N_FEAT, HID), lambda i: (0, 0)),
            pl.BlockSpec((NC, BLK, 16), lambda i: (0, i, 0)),
        ],
        out_specs=[
            pl.BlockSpec((BLK, HID), lambda i: (i, 0)),
            pl.BlockSpec((NC, BLK, HID), lambda i: (0, i, 0)),
        ],
        out_shape=[
            jax.ShapeDtypeStruct((NP, HID), F32),
            jax.ShapeDtypeStruct((NC, NP, HID), F32),
        ],
    )(xp, W1, deg_p)


def _mid_body(agg_ref, h1_ref, deg_ref, b1_ref, w2_ref, h2_ref, g2_ref):
    dv = _dinv_col(deg_ref)
    agg = agg_ref[0] + agg_ref[1]
    out1 = jnp.maximum(dv * agg + dv * dv * h1_ref[...] + b1_ref[...], 0.0)
    h2 = jnp.dot(out1, w2_ref[...], preferred_element_type=F32)
    h2_ref[...] = h2
    g2 = h2 * dv
    g2_ref[0] = g2[:, 0:HID]
    g2_ref[1] = g2[:, HID:N_FEAT]


def _tc_mid(agg1, H1, deg_p, b1, W2):
    return pl.pallas_call(
        _mid_body,
        grid=(NP // BLK,),
        in_specs=[
            pl.BlockSpec((NC, BLK, HID), lambda i: (0, i, 0)),
            pl.BlockSpec((BLK, HID), lambda i: (i, 0)),
            pl.BlockSpec((NC, BLK, 16), lambda i: (0, i, 0)),
            pl.BlockSpec((1, HID), lambda i: (0, 0)),
            pl.BlockSpec((HID, N_FEAT), lambda i: (0, 0)),
        ],
        out_specs=[
            pl.BlockSpec((BLK, N_FEAT), lambda i: (i, 0)),
            pl.BlockSpec((NC, BLK, HID), lambda i: (0, i, 0)),
        ],
        out_shape=[
            jax.ShapeDtypeStruct((NP, N_FEAT), F32),
            jax.ShapeDtypeStruct((NC, NP, HID), F32),
        ],
    )(agg1, H1, deg_p, b1, W2)


def _post_body(agg_ref, h2_ref, deg_ref, b2_ref, out_ref):
    dv = _dinv_col(deg_ref)
    agg = jnp.concatenate([agg_ref[0], agg_ref[1]], axis=1)
    out_ref[...] = jnp.maximum(dv * agg + dv * dv * h2_ref[...] + b2_ref[...], 0.0)


def _tc_post(agg2, H2, deg_p, b2):
    return pl.pallas_call(
        _post_body,
        grid=(NP // BLK,),
        in_specs=[
            pl.BlockSpec((NC, BLK, HID), lambda i: (0, i, 0)),
            pl.BlockSpec((BLK, N_FEAT), lambda i: (i, 0)),
            pl.BlockSpec((NC, BLK, 16), lambda i: (0, i, 0)),
            pl.BlockSpec((1, N_FEAT), lambda i: (0, 0)),
        ],
        out_specs=pl.BlockSpec((BLK, N_FEAT), lambda i: (i, 0)),
        out_shape=jax.ShapeDtypeStruct((NP, N_FEAT), F32),
    )(agg2, H2, deg_p, b2)


def _lstm_in_body(x_ref, w_ref, bi_ref, bh_ref, out_ref):
    k = pl.program_id(0)

    @pl.when(k == 0)
    def _init():
        out_ref[...] = jnp.zeros_like(out_ref)

    out_ref[...] += jnp.dot(x_ref[...], w_ref[...], preferred_element_type=F32)

    @pl.when(k == pl.num_programs(0) - 1)
    def _bias():
        out_ref[...] += bi_ref[...] + bh_ref[...]


def _tc_lstm_in(xl, W_ihT, bi, bh):
    kb = 512
    return pl.pallas_call(
        _lstm_in_body,
        grid=(LSTM_IN // kb,),
        in_specs=[
            pl.BlockSpec((104, kb), lambda k: (0, k)),
            pl.BlockSpec((kb, 4 * LSTM_H), lambda k: (k, 0)),
            pl.BlockSpec((1, 4 * LSTM_H), lambda k: (0, 0)),
            pl.BlockSpec((1, 4 * LSTM_H), lambda k: (0, 0)),
        ],
        out_specs=pl.BlockSpec((104, 4 * LSTM_H), lambda k: (0, 0)),
        out_shape=jax.ShapeDtypeStruct((104, 4 * LSTM_H), F32),
    )(xl, W_ihT, bi, bh)


def _lstm_fc_body(g_ref, whh_ref, fc1w_ref, fc1b_ref, fc2w_ref, fc2b_ref, out_ref):
    whh = whh_ref[...]
    h = jnp.zeros((BATCH, LSTM_H), F32)
    cst = jnp.zeros((BATCH, LSTM_H), F32)
    hs = []
    for t in range(WIN_IN):
        gt = jnp.concatenate(
            [g_ref[pl.ds(b * WIN_IN + t, 1), :] for b in range(BATCH)], axis=0)
        gates = gt + jnp.dot(h, whh, preferred_element_type=F32)
        i_g = jax.nn.sigmoid(gates[:, 0:LSTM_H])
        f_g = jax.nn.sigmoid(gates[:, LSTM_H:2 * LSTM_H])
        g_g = jnp.tanh(gates[:, 2 * LSTM_H:3 * LSTM_H])
        o_g = jax.nn.sigmoid(gates[:, 3 * LSTM_H:4 * LSTM_H])
        cst = f_g * cst + i_g * g_g
        h = o_g * jnp.tanh(cst)
        if t >= WIN_IN - WIN_OUT:
            hs.append(h)
    hcat = jnp.concatenate(hs, axis=0)
    z = jnp.maximum(
        jnp.dot(hcat, fc1w_ref[...], preferred_element_type=F32) + fc1b_ref[...], 0.0)
    out_ref[...] = jnp.dot(z, fc2w_ref[...], preferred_element_type=F32) + fc2b_ref[...]


def _tc_lstm_fc(G, W_hhT, fc1_W, fc1_b, fc2_W, fc2_b):
    return pl.pallas_call(
        _lstm_fc_body,
        out_shape=jax.ShapeDtypeStruct((BATCH * WIN_OUT, NCLS), F32),
    )(G, W_hhT, fc1_W, fc1_b, fc2_W, fc2_b)


def kernel(x, edge_index, W1, b1, W2, b2, W_ih, W_hh, b_ih, b_hh,
           fc1_W, fc1_b, fc2_W, fc2_b):
    xp = jnp.pad(x, ((0, NP - N_TOTAL), (0, 0)))
    pad = jnp.full((E_PAD - N_EDGES,), N_TOTAL, jnp.int32)
    src_f = jnp.concatenate([edge_index[0], pad])
    dst_f = jnp.concatenate([edge_index[1], pad])
    srcp = src_f.reshape(NW, K_CH, CHUNK)
    dstp = dst_f.reshape(NW, K_CH, CHUNK)
    srcq = src_f.reshape(NS, K2, CHUNK)
    dstq = dst_f.reshape(NS, K2, CHUNK)

    deg_p = _sc_degree(dstp)

    H1, g1 = _tc_prep(xp, W1, deg_p)
    agg1 = _sc_aggregate(g1, srcp, dstp, HID)
    H2, g2h = _tc_mid(agg1, H1, deg_p, b1.reshape(1, HID), W2)
    agg2 = _sc_aggregate_feat(g2h, srcq, dstq)
    out2 = _tc_post(agg2, H2, deg_p, b2.reshape(1, N_FEAT))

    xl = out2[:N_TOTAL].reshape(BATCH * WIN_IN, LSTM_IN)
    xl = jnp.pad(xl, ((0, 4), (0, 0)))
    G = _tc_lstm_in(xl, W_ih.T, b_ih.reshape(1, -1), b_hh.reshape(1, -1))
    out_t = _tc_lstm_fc(G, W_hh.T, fc1_W, fc1_b.reshape(1, -1),
                        fc2_W, fc2_b.reshape(1, -1))
    return (out_t.reshape(WIN_OUT, BATCH, NCLS)
            .transpose(1, 0, 2).reshape(BATCH * WIN_OUT, NCLS))


# speedup vs baseline: 1.0158x; 1.0003x over previous
import functools

import jax
import jax.numpy as jnp
from jax import lax
from jax.experimental import pallas as pl
from jax.experimental.pallas import tpu as pltpu
from jax.experimental.pallas import tpu_sc as plsc

F32 = jnp.float32

N_TOTAL = 10000
NP = 10240
N_FEAT = 128
HID = 64
N_EDGES = 320000
NC, NS = 2, 16
NW = NC * NS
CHUNK = 128
K_CH = 80
E_PAD = NW * K_CH * CHUNK
RPT = NP // NS
BATCH, WIN_IN, WIN_OUT = 5, 20, 5
LSTM_IN = 12800
LSTM_H = 32
NCLS = 10
BLK = 1024
NSLOT = 5
NAHEAD = 4


def _agg_pipeline(g_hbm, src_v, dst_v, bufs, gsems, ssems, acc, K):
    for b in range(NAHEAD):
        pltpu.async_copy(g_hbm.at[src_v.at[b]], bufs[b], gsems[b])

    def body(gi, carry):
        for b in range(NSLOT):
            j = gi * NSLOT + b
            b2 = (b + NAHEAD) % NSLOT
            pltpu.make_async_copy(g_hbm.at[src_v.at[j]], bufs[b], gsems[b]).wait()
            pltpu.async_copy(bufs[b], acc.at[dst_v.at[j]], ssems[b], add=True)

            @pl.when(j + NAHEAD < K)
            def _next():
                @pl.when(j >= NSLOT - NAHEAD)
                def _drain_prev():
                    pltpu.make_async_copy(
                        bufs[b2], acc.at[dst_v.at[j + NAHEAD - NSLOT]],
                        ssems[b2]).wait()
                pltpu.async_copy(g_hbm.at[src_v.at[j + NAHEAD]], bufs[b2], gsems[b2])
        return carry

    lax.fori_loop(0, K // NSLOT, body, 0)
    for j in range(K - NSLOT, K):
        b = j % NSLOT
        pltpu.make_async_copy(bufs[b], acc.at[dst_v.at[j]], ssems[b]).wait()

@functools.cache
def _deg_call():
    mesh = plsc.VectorSubcoreMesh(core_axis_name="c", subcore_axis_name="s")

    @functools.partial(
        pl.kernel,
        mesh=mesh,
        compiler_params=pltpu.CompilerParams(use_tc_tiling_on_sc=False),
        out_type=jax.ShapeDtypeStruct((NC, NP, 16), F32),
        scratch_types=[
            pltpu.VMEM((K_CH, CHUNK), jnp.int32),
            pltpu.VMEM((CHUNK, 16), F32),
            pltpu.VMEM((16, 16), F32),
            pltpu.VMEM_SHARED((NP, 16), F32),
        ],
    )
    def deg_kernel(dst_hbm, out_hbm, dst_v, ones_v, zt_v, acc):
        c = lax.axis_index("c")
        s = lax.axis_index("s")
        wid = s * NC + c
        pltpu.sync_copy(dst_hbm.at[wid], dst_v)
        for r in range(16):
            zt_v[r, :] = jnp.zeros((16,), F32)
        for r in range(CHUNK):
            ones_v[r, :] = jnp.ones((16,), F32)

        def zbody(i, carry):
            pltpu.sync_copy(zt_v, acc.at[pl.ds(s * RPT + i * 16, 16)])
            return carry

        lax.fori_loop(0, RPT // 16, zbody, 0)
        plsc.subcore_barrier()

        def sbody(j, carry):
            pltpu.sync_copy(ones_v, acc.at[dst_v.at[j]], add=True)
            return carry

        lax.fori_loop(0, K_CH, sbody, 0)
        plsc.subcore_barrier()
        pltpu.sync_copy(acc.at[pl.ds(s * RPT, RPT)],
                        out_hbm.at[c, pl.ds(s * RPT, RPT)])

    return deg_kernel


@functools.cache
def _agg_call(D):
    mesh = plsc.VectorSubcoreMesh(core_axis_name="c", subcore_axis_name="s")

    @functools.partial(
        pl.kernel,
        mesh=mesh,
        compiler_params=pltpu.CompilerParams(use_tc_tiling_on_sc=False),
        out_type=jax.ShapeDtypeStruct((NC, NP, D), F32),
        scratch_types=[
            pltpu.VMEM((K_CH, CHUNK), jnp.int32),
            pltpu.VMEM((K_CH, CHUNK), jnp.int32),
            pltpu.VMEM((NSLOT * CHUNK, D), F32),
            pltpu.VMEM((16, D), F32),
            pltpu.VMEM_SHARED((NP, D), F32),
        ] + [pltpu.SemaphoreType.DMA] * (2 * NSLOT),
    )
    def agg_kernel(g_hbm, src_hbm, dst_hbm, out_hbm,
                   src_v, dst_v, bufs_v, zt_v, acc, *sems):
        c = lax.axis_index("c")
        s = lax.axis_index("s")
        wid = s * NC + c
        pltpu.sync_copy(src_hbm.at[wid], src_v)
        pltpu.sync_copy(dst_hbm.at[wid], dst_v)
        for r in range(16):
            for q in range(D // 16):
                zt_v[r, pl.ds(q * 16, 16)] = jnp.zeros((16,), F32)

        def zbody(i, carry):
            pltpu.sync_copy(zt_v, acc.at[pl.ds(s * RPT + i * 16, 16)])
            return carry

        lax.fori_loop(0, RPT // 16, zbody, 0)
        plsc.subcore_barrier()

        bufs = [bufs_v.at[pl.ds(b * CHUNK, CHUNK)] for b in range(NSLOT)]
        _agg_pipeline(g_hbm.at[c], src_v, dst_v, bufs,
                      sems[:NSLOT], sems[NSLOT:], acc, K_CH)
        plsc.subcore_barrier()
        pltpu.sync_copy(acc.at[pl.ds(s * RPT, RPT)],
                        out_hbm.at[c, pl.ds(s * RPT, RPT)])

    return agg_kernel


K2 = 160


@functools.cache
def _agg_feat_call():
    mesh = plsc.VectorSubcoreMesh(core_axis_name="c", subcore_axis_name="s")

    @functools.partial(
        pl.kernel,
        mesh=mesh,
        compiler_params=pltpu.CompilerParams(use_tc_tiling_on_sc=False),
        out_type=jax.ShapeDtypeStruct((NC, NP, HID), F32),
        scratch_types=[
            pltpu.VMEM((K2, CHUNK), jnp.int32),
            pltpu.VMEM((K2, CHUNK), jnp.int32),
            pltpu.VMEM((NSLOT * CHUNK, HID), F32),
            pltpu.VMEM((16, HID), F32),
            pltpu.VMEM_SHARED((NP, HID), F32),
        ] + [pltpu.SemaphoreType.DMA] * (2 * NSLOT),
    )
    def agg2_kernel(g_hbm, src_hbm, dst_hbm, out_hbm,
                    src_v, dst_v, bufs_v, zt_v, acc, *sems):
        c = lax.axis_index("c")
        s = lax.axis_index("s")
        pltpu.sync_copy(src_hbm.at[s], src_v)
        pltpu.sync_copy(dst_hbm.at[s], dst_v)
        for r in range(16):
            for q in range(HID // 16):
                zt_v[r, pl.ds(q * 16, 16)] = jnp.zeros((16,), F32)

        def zbody(i, carry):
            pltpu.sync_copy(zt_v, acc.at[pl.ds(s * RPT + i * 16, 16)])
            return carry

        lax.fori_loop(0, RPT // 16, zbody, 0)
        plsc.subcore_barrier()

        bufs = [bufs_v.at[pl.ds(b * CHUNK, CHUNK)] for b in range(NSLOT)]
        _agg_pipeline(g_hbm.at[c], src_v, dst_v, bufs,
                      sems[:NSLOT], sems[NSLOT:], acc, K2)
        plsc.subcore_barrier()
        pltpu.sync_copy(acc.at[pl.ds(s * RPT, RPT)],
                        out_hbm.at[c, pl.ds(s * RPT, RPT)])

    return agg2_kernel


def _sc_degree(dstp):
    return _deg_call()(dstp)


def _sc_aggregate(g, srcp, dstp, D):
    return _agg_call(D)(g, srcp, dstp)


def _sc_aggregate_feat(g2d, srcq, dstq):
    return _agg_feat_call()(g2d, srcq, dstq)


def _dinv_col(deg_ref):
    d = deg_ref[0] + deg_ref[1]
    return lax.rsqrt(d[:, 0:1] + 1.0)


def _prep_body(x_ref, w_ref, deg_ref, h1_ref, g1_ref):
    h = jnp.dot(x_ref[...], w_ref[...], preferred_element_type=F32)
    h1_ref[...] = h
    g1 = h * _dinv_col(deg_ref)
    g1_ref[0] = g1
    g1_ref[1] = g1


def _tc_prep(xp, W1, deg_p):
    return pl.pallas_call(
        _prep_body,
        grid=(NP // BLK,),
        in_specs=[
            pl.BlockSpec((BLK, N_FEAT), lambda i: (i, 0)),
            pl.BlockSpec((N_FEAT, HID), lambda i: (0, 0)),
            pl.BlockSpec((NC, BLK, 16), lambda i: (0, i, 0)),
        ],
        out_specs=[
            pl.BlockSpec((BLK, HID), lambda i: (i, 0)),
            pl.BlockSpec((NC, BLK, HID), lambda i: (0, i, 0)),
        ],
        out_shape=[
            jax.ShapeDtypeStruct((NP, HID), F32),
            jax.ShapeDtypeStruct((NC, NP, HID), F32),
        ],
    )(xp, W1, deg_p)


def _mid_body(agg_ref, h1_ref, deg_ref, b1_ref, w2_ref, h2_ref, g2_ref):
    dv = _dinv_col(deg_ref)
    agg = agg_ref[0] + agg_ref[1]
    out1 = jnp.maximum(dv * agg + dv * dv * h1_ref[...] + b1_ref[...], 0.0)
    h2 = jnp.dot(out1, w2_ref[...], preferred_element_type=F32)
    h2_ref[...] = h2
    g2 = h2 * dv
    g2_ref[0] = g2[:, 0:HID]
    g2_ref[1] = g2[:, HID:N_FEAT]


def _tc_mid(agg1, H1, deg_p, b1, W2):
    return pl.pallas_call(
        _mid_body,
        grid=(NP // BLK,),
        in_specs=[
            pl.BlockSpec((NC, BLK, HID), lambda i: (0, i, 0)),
            pl.BlockSpec((BLK, HID), lambda i: (i, 0)),
            pl.BlockSpec((NC, BLK, 16), lambda i: (0, i, 0)),
            pl.BlockSpec((1, HID), lambda i: (0, 0)),
            pl.BlockSpec((HID, N_FEAT), lambda i: (0, 0)),
        ],
        out_specs=[
            pl.BlockSpec((BLK, N_FEAT), lambda i: (i, 0)),
            pl.BlockSpec((NC, BLK, HID), lambda i: (0, i, 0)),
        ],
        out_shape=[
            jax.ShapeDtypeStruct((NP, N_FEAT), F32),
            jax.ShapeDtypeStruct((NC, NP, HID), F32),
        ],
    )(agg1, H1, deg_p, b1, W2)


def _post_body(agg_ref, h2_ref, deg_ref, b2_ref, out_ref):
    dv = _dinv_col(deg_ref)
    agg = jnp.concatenate([agg_ref[0], agg_ref[1]], axis=1)
    out_ref[...] = jnp.maximum(dv * agg + dv * dv * h2_ref[...] + b2_ref[...], 0.0)


def _tc_post(agg2, H2, deg_p, b2):
    return pl.pallas_call(
        _post_body,
        grid=(NP // BLK,),
        in_specs=[
            pl.BlockSpec((NC, BLK, HID), lambda i: (0, i, 0)),
            pl.BlockSpec((BLK, N_FEAT), lambda i: (i, 0)),
            pl.BlockSpec((NC, BLK, 16), lambda i: (0, i, 0)),
            pl.BlockSpec((1, N_FEAT), lambda i: (0, 0)),
        ],
        out_specs=pl.BlockSpec((BLK, N_FEAT), lambda i: (i, 0)),
        out_shape=jax.ShapeDtypeStruct((NP, N_FEAT), F32),
    )(agg2, H2, deg_p, b2)


def _lstm_in_body(x_ref, w_ref, bi_ref, bh_ref, out_ref):
    k = pl.program_id(0)

    @pl.when(k == 0)
    def _init():
        out_ref[...] = jnp.zeros_like(out_ref)

    out_ref[...] += jnp.dot(x_ref[...], w_ref[...], preferred_element_type=F32)

    @pl.when(k == pl.num_programs(0) - 1)
    def _bias():
        out_ref[...] += bi_ref[...] + bh_ref[...]


def _tc_lstm_in(xl, W_ihT, bi, bh):
    kb = 512
    return pl.pallas_call(
        _lstm_in_body,
        grid=(LSTM_IN // kb,),
        in_specs=[
            pl.BlockSpec((104, kb), lambda k: (0, k)),
            pl.BlockSpec((kb, 4 * LSTM_H), lambda k: (k, 0)),
            pl.BlockSpec((1, 4 * LSTM_H), lambda k: (0, 0)),
            pl.BlockSpec((1, 4 * LSTM_H), lambda k: (0, 0)),
        ],
        out_specs=pl.BlockSpec((104, 4 * LSTM_H), lambda k: (0, 0)),
        out_shape=jax.ShapeDtypeStruct((104, 4 * LSTM_H), F32),
    )(xl, W_ihT, bi, bh)


def _lstm_fc_body(g_ref, whh_ref, fc1w_ref, fc1b_ref, fc2w_ref, fc2b_ref, out_ref):
    whh = whh_ref[...]
    h = jnp.zeros((BATCH, LSTM_H), F32)
    cst = jnp.zeros((BATCH, LSTM_H), F32)
    hs = []
    for t in range(WIN_IN):
        gt = jnp.concatenate(
            [g_ref[pl.ds(b * WIN_IN + t, 1), :] for b in range(BATCH)], axis=0)
        gates = gt + jnp.dot(h, whh, preferred_element_type=F32)
        i_g = jax.nn.sigmoid(gates[:, 0:LSTM_H])
        f_g = jax.nn.sigmoid(gates[:, LSTM_H:2 * LSTM_H])
        g_g = jnp.tanh(gates[:, 2 * LSTM_H:3 * LSTM_H])
        o_g = jax.nn.sigmoid(gates[:, 3 * LSTM_H:4 * LSTM_H])
        cst = f_g * cst + i_g * g_g
        h = o_g * jnp.tanh(cst)
        if t >= WIN_IN - WIN_OUT:
            hs.append(h)
    hcat = jnp.concatenate(hs, axis=0)
    z = jnp.maximum(
        jnp.dot(hcat, fc1w_ref[...], preferred_element_type=F32) + fc1b_ref[...], 0.0)
    out_ref[...] = jnp.dot(z, fc2w_ref[...], preferred_element_type=F32) + fc2b_ref[...]


def _tc_lstm_fc(G, W_hhT, fc1_W, fc1_b, fc2_W, fc2_b):
    return pl.pallas_call(
        _lstm_fc_body,
        out_shape=jax.ShapeDtypeStruct((BATCH * WIN_OUT, NCLS), F32),
    )(G, W_hhT, fc1_W, fc1_b, fc2_W, fc2_b)


def kernel(x, edge_index, W1, b1, W2, b2, W_ih, W_hh, b_ih, b_hh,
           fc1_W, fc1_b, fc2_W, fc2_b):
    xp = jnp.pad(x, ((0, NP - N_TOTAL), (0, 0)))
    pad = jnp.full((E_PAD - N_EDGES,), N_TOTAL, jnp.int32)
    src_f = jnp.concatenate([edge_index[0], pad])
    dst_f = jnp.concatenate([edge_index[1], pad])
    srcp = src_f.reshape(NW, K_CH, CHUNK)
    dstp = dst_f.reshape(NW, K_CH, CHUNK)
    srcq = src_f.reshape(NS, K2, CHUNK)
    dstq = dst_f.reshape(NS, K2, CHUNK)

    deg_p = _sc_degree(dstp)

    H1, g1 = _tc_prep(xp, W1, deg_p)
    agg1 = _sc_aggregate(g1, srcp, dstp, HID)
    H2, g2h = _tc_mid(agg1, H1, deg_p, b1.reshape(1, HID), W2)
    agg2 = _sc_aggregate_feat(g2h, srcq, dstq)
    out2 = _tc_post(agg2, H2, deg_p, b2.reshape(1, N_FEAT))

    xl = out2[:N_TOTAL].reshape(BATCH * WIN_IN, LSTM_IN)
    xl = jnp.pad(xl, ((0, 4), (0, 0)))
    G = _tc_lstm_in(xl, W_ih.T, b_ih.reshape(1, -1), b_hh.reshape(1, -1))
    out_t = _tc_lstm_fc(G, W_hh.T, fc1_W, fc1_b.reshape(1, -1),
                        fc2_W, fc2_b.reshape(1, -1))
    return (out_t.reshape(WIN_OUT, BATCH, NCLS)
            .transpose(1, 0, 2).reshape(BATCH * WIN_OUT, NCLS))
